# baseline (speedup 1.0000x reference)
import functools
import math

import jax
import jax.numpy as jnp
from jax import lax
from jax.experimental import pallas as pl
from jax.experimental.pallas import tpu as pltpu
from jax.experimental.pallas import tpu_sc as plsc

N, C, H, W_DIM = 8, 256, 100, 252
HIDDEN = 256
HW = H * W_DIM
K = max(int(math.ceil(HW * 0.2)), 1)
LANES = 16
SEG = HW // LANES
NBINS = 256
SCORE_BLK = 2520
MASK_BLK = 3150


SCORE_G = 8
SCORE_CHUNK = SEG


def _score_body(x_ref, w1_ref, b1_ref, w2_ref, b2_ref, s_ref):
  w116 = w1_ref[...].astype(jnp.bfloat16)
  w216 = w2_ref[...].astype(jnp.bfloat16)
  for i in range(SCORE_G):
    xs16 = x_ref[0, :, i, :].astype(jnp.bfloat16)
    hdn = lax.dot_general(w116, xs16, (((0,), (0,)), ((), ())),
                          preferred_element_type=jnp.float32)
    hdn16 = jnp.maximum(hdn + b1_ref[...], 0.0).astype(jnp.bfloat16)
    z = lax.dot_general(w216, hdn16, (((0,), (0,)), ((), ())),
                        preferred_element_type=jnp.float32)
    s_ref[0, pl.ds(i, 1), :] = jax.nn.sigmoid(z + b2_ref[...])


def _scores(xf, W1, b1, W2, b2):
  x4 = xf.reshape(N, C, HW // SCORE_CHUNK, SCORE_CHUNK)
  return pl.pallas_call(
      _score_body,
      grid=(N, HW // (SCORE_G * SCORE_CHUNK)),
      in_specs=[
          pl.BlockSpec((1, C, SCORE_G, SCORE_CHUNK), lambda n, j: (n, 0, j, 0)),
          pl.BlockSpec((C, HIDDEN), lambda n, j: (0, 0)),
          pl.BlockSpec((HIDDEN, 1), lambda n, j: (0, 0)),
          pl.BlockSpec((HIDDEN, 1), lambda n, j: (0, 0)),
          pl.BlockSpec((1, 1), lambda n, j: (0, 0)),
      ],
      out_specs=pl.BlockSpec((1, SCORE_G, SCORE_CHUNK), lambda n, j: (n, j, 0)),
      out_shape=jax.ShapeDtypeStruct((N, HW // SCORE_CHUNK, SCORE_CHUNK),
                                     jnp.float32),
  )(x4, W1, b1.reshape(HIDDEN, 1), W2, b2.reshape(1, 1))


SC_CORES = 2
SC_SUBCORES = 16


def _sort_body(sbits_hbm, tk_hbm, wmap_hbm, ka, va, kb, vb, offs):
  wid = lax.axis_index("s") * SC_CORES + lax.axis_index("c")

  @pl.when(wid < N)
  def _():
    a = wid
    lane = lax.iota(jnp.int32, LANES)
    seg_base = lane * SEG
    ones16 = jnp.ones((LANES,), jnp.int32)
    zero16 = jnp.zeros((LANES,), jnp.int32)

    pltpu.sync_copy(sbits_hbm.at[pl.ds(a * HW, HW)], kb)

    def _init(t, _):
      sl = pl.ds(t * LANES, LANES)
      ka[sl] = ~kb[sl]
      va[sl] = t * LANES + lane
      return 0
    lax.fori_loop(0, HW // LANES, _init, 0)

    bufs = [(ka, va, kb, vb), (kb, vb, ka, va),
            (ka, va, kb, vb), (kb, vb, ka, va)]
    for p, (kin, vin, kout, vout) in enumerate(bufs):
      shift = 8 * p

      def _zero(i, _):
        offs[pl.ds(i * LANES, LANES)] = zero16
        return 0
      lax.fori_loop(0, NBINS, _zero, 0)

      def _hist(t, _, kin=kin, shift=shift):
        k16 = plsc.load_gather(kin, [seg_base + t])
        d = lax.shift_right_logical(k16, shift) & 0xFF
        plsc.addupdate_scatter(offs, [d * LANES + lane], ones16)
        return 0
      lax.fori_loop(0, SEG, _hist, 0)

      def _scan(d, run):
        sl = pl.ds(d * LANES, LANES)
        v = offs[sl]
        offs[sl] = run + plsc.cumsum(v) - v
        return run + jnp.sum(v)
      lax.fori_loop(0, NBINS, _scan, jnp.int32(0))

      def _place(t, _, kin=kin, vin=vin, kout=kout, vout=vout, shift=shift):
        idx = seg_base + t
        k16 = plsc.load_gather(kin, [idx])
        v16 = plsc.load_gather(vin, [idx])
        d = lax.shift_right_logical(k16, shift) & 0xFF
        addr = d * LANES + lane
        pos = plsc.load_gather(offs, [addr])
        plsc.store_scatter(kout, [pos], k16)
        plsc.store_scatter(vout, [pos], v16)
        plsc.addupdate_scatter(offs, [addr], ones16)
        return 0
      lax.fori_loop(0, SEG, _place, 0)

    def _vals(j, _):
      sl = pl.ds(j * LANES, LANES)
      kb[sl] = ~ka[sl]
      return 0
    lax.fori_loop(0, K // LANES, _vals, 0)
    pltpu.sync_copy(kb.at[pl.ds(0, K)], tk_hbm.at[pl.ds(a * K, K)])

    def _wzero(t, _):
      vb[pl.ds(t * LANES, LANES)] = zero16
      return 0
    lax.fori_loop(0, HW // LANES, _wzero, 0)

    def _wscat(j, _):
      sl = pl.ds(j * LANES, LANES)
      plsc.store_scatter(vb, [va[sl]], ~ka[sl])
      return 0
    lax.fori_loop(0, K // LANES, _wscat, 0)
    pltpu.sync_copy(vb, wmap_hbm.at[pl.ds(a * HW, HW)])


def _topk(sbits):
  mesh = plsc.VectorSubcoreMesh(core_axis_name="c", subcore_axis_name="s",
                                num_cores=SC_CORES, num_subcores=SC_SUBCORES)
  f = pl.kernel(
      _sort_body,
      out_type=(jax.ShapeDtypeStruct((N * K,), jnp.int32),
                jax.ShapeDtypeStruct((N * HW,), jnp.int32)),
      mesh=mesh,
      compiler_params=pltpu.CompilerParams(needs_layout_passes=False),
      scratch_types=[
          pltpu.VMEM((HW,), jnp.int32),
          pltpu.VMEM((HW,), jnp.int32),
          pltpu.VMEM((HW,), jnp.int32),
          pltpu.VMEM((HW,), jnp.int32),
          pltpu.VMEM((NBINS * LANES,), jnp.int32),
      ],
  )
  return f(sbits)


MASK_CBLK = 32


def _mask_body(x_ref, w_ref, o_ref):
  o_ref[0] = x_ref[0] * w_ref[...]


def _mask_mul(x, wmap_hw):
  return pl.pallas_call(
      _mask_body,
      grid=(N, C // MASK_CBLK),
      in_specs=[
          pl.BlockSpec((1, MASK_CBLK, H, W_DIM), lambda n, j: (n, j, 0, 0)),
          pl.BlockSpec((1, H, W_DIM), lambda n, j: (n, 0, 0)),
      ],
      out_specs=pl.BlockSpec((1, MASK_CBLK, H, W_DIM), lambda n, j: (n, j, 0, 0)),
      out_shape=jax.ShapeDtypeStruct((N, C, H, W_DIM), jnp.float32),
  )(x, wmap_hw)


def kernel(x, W1, b1, W2, b2):
  xf = x.reshape(N, C, HW)
  scores = _scores(xf, W1, b1, W2, b2)
  sbits = lax.bitcast_convert_type(scores, jnp.int32).reshape(N * HW)
  tk_bits, w_bits = _topk(sbits)
  topk_vals = lax.bitcast_convert_type(tk_bits.reshape(N, K), jnp.float32)
  wmap_hw = lax.bitcast_convert_type(w_bits.reshape(N, H, W_DIM), jnp.float32)
  sparse = _mask_mul(x, wmap_hw)
  return sparse, topk_vals


# speedup vs baseline: 1.0659x; 1.0659x over previous
import functools
import math

import jax
import jax.numpy as jnp
from jax import lax
from jax.experimental import pallas as pl
from jax.experimental.pallas import tpu as pltpu
from jax.experimental.pallas import tpu_sc as plsc

N, C, H, W_DIM = 8, 256, 100, 252
HIDDEN = 256
HW = H * W_DIM
K = max(int(math.ceil(HW * 0.2)), 1)
LANES = 16
SEG = HW // LANES
NBINS = 256
SCORE_BLK = 2520
MASK_BLK = 3150


SCORE_HB = 8


def _score_body(x_ref, w1_ref, b1_ref, w2_ref, b2_ref, s_ref):
  w116 = w1_ref[...].astype(jnp.bfloat16)
  w216 = w2_ref[...].astype(jnp.bfloat16)
  for i in range(SCORE_HB):
    xs16 = x_ref[0, :, i, :].astype(jnp.bfloat16)
    hdn = lax.dot_general(w116, xs16, (((0,), (0,)), ((), ())),
                          preferred_element_type=jnp.float32)
    hdn16 = jnp.maximum(hdn + b1_ref[...], 0.0).astype(jnp.bfloat16)
    z = lax.dot_general(w216, hdn16, (((0,), (0,)), ((), ())),
                        preferred_element_type=jnp.float32)
    s_ref[0, pl.ds(i, 1), :] = jax.nn.sigmoid(z + b2_ref[...])


def _scores(x, W1, b1, W2, b2):
  return pl.pallas_call(
      _score_body,
      grid=(N, (H + SCORE_HB - 1) // SCORE_HB),
      in_specs=[
          pl.BlockSpec((1, C, SCORE_HB, W_DIM), lambda n, j: (n, 0, j, 0)),
          pl.BlockSpec((C, HIDDEN), lambda n, j: (0, 0)),
          pl.BlockSpec((HIDDEN, 1), lambda n, j: (0, 0)),
          pl.BlockSpec((HIDDEN, 1), lambda n, j: (0, 0)),
          pl.BlockSpec((1, 1), lambda n, j: (0, 0)),
      ],
      out_specs=pl.BlockSpec((1, SCORE_HB, W_DIM), lambda n, j: (n, j, 0)),
      out_shape=jax.ShapeDtypeStruct((N, H, W_DIM), jnp.float32),
  )(x, W1, b1.reshape(HIDDEN, 1), W2, b2.reshape(1, 1))


SC_CORES = 2
SC_SUBCORES = 16


def _sort_body(sbits_hbm, tk_hbm, wmap_hbm, ka, va, kb, vb, offs):
  wid = lax.axis_index("s") * SC_CORES + lax.axis_index("c")

  @pl.when(wid < N)
  def _():
    a = wid
    lane = lax.iota(jnp.int32, LANES)
    seg_base = lane * SEG
    ones16 = jnp.ones((LANES,), jnp.int32)
    zero16 = jnp.zeros((LANES,), jnp.int32)

    pltpu.sync_copy(sbits_hbm.at[pl.ds(a * HW, HW)], kb)

    def _init(t, _):
      sl = pl.ds(t * LANES, LANES)
      ka[sl] = ~kb[sl]
      va[sl] = t * LANES + lane
      return 0
    lax.fori_loop(0, HW // LANES, _init, 0)

    bufs = [(ka, va, kb, vb), (kb, vb, ka, va),
            (ka, va, kb, vb), (kb, vb, ka, va)]
    for p, (kin, vin, kout, vout) in enumerate(bufs):
      shift = 8 * p

      def _zero(i, _):
        offs[pl.ds(i * LANES, LANES)] = zero16
        return 0
      lax.fori_loop(0, NBINS, _zero, 0)

      def _hist(t, _, kin=kin, shift=shift):
        k16 = plsc.load_gather(kin, [seg_base + t])
        d = lax.shift_right_logical(k16, shift) & 0xFF
        plsc.addupdate_scatter(offs, [d * LANES + lane], ones16)
        return 0
      lax.fori_loop(0, SEG, _hist, 0)

      def _scan(d, run):
        sl = pl.ds(d * LANES, LANES)
        v = offs[sl]
        offs[sl] = run + plsc.cumsum(v) - v
        return run + jnp.sum(v)
      lax.fori_loop(0, NBINS, _scan, jnp.int32(0))

      def _place(t, _, kin=kin, vin=vin, kout=kout, vout=vout, shift=shift):
        idx = seg_base + t
        k16 = plsc.load_gather(kin, [idx])
        v16 = plsc.load_gather(vin, [idx])
        d = lax.shift_right_logical(k16, shift) & 0xFF
        addr = d * LANES + lane
        pos = plsc.load_gather(offs, [addr])
        plsc.store_scatter(kout, [pos], k16)
        plsc.store_scatter(vout, [pos], v16)
        plsc.addupdate_scatter(offs, [addr], ones16)
        return 0
      lax.fori_loop(0, SEG, _place, 0)

    def _vals(j, _):
      sl = pl.ds(j * LANES, LANES)
      kb[sl] = ~ka[sl]
      return 0
    lax.fori_loop(0, K // LANES, _vals, 0)
    pltpu.sync_copy(kb.at[pl.ds(0, K)], tk_hbm.at[pl.ds(a * K, K)])

    def _wzero(t, _):
      vb[pl.ds(t * LANES, LANES)] = zero16
      return 0
    lax.fori_loop(0, HW // LANES, _wzero, 0)

    def _wscat(j, _):
      sl = pl.ds(j * LANES, LANES)
      plsc.store_scatter(vb, [va[sl]], ~ka[sl])
      return 0
    lax.fori_loop(0, K // LANES, _wscat, 0)
    pltpu.sync_copy(vb, wmap_hbm.at[pl.ds(a * HW, HW)])


def _topk(sbits):
  mesh = plsc.VectorSubcoreMesh(core_axis_name="c", subcore_axis_name="s",
                                num_cores=SC_CORES, num_subcores=SC_SUBCORES)
  f = pl.kernel(
      _sort_body,
      out_type=(jax.ShapeDtypeStruct((N * K,), jnp.int32),
                jax.ShapeDtypeStruct((N * HW,), jnp.int32)),
      mesh=mesh,
      compiler_params=pltpu.CompilerParams(needs_layout_passes=False),
      scratch_types=[
          pltpu.VMEM((HW,), jnp.int32),
          pltpu.VMEM((HW,), jnp.int32),
          pltpu.VMEM((HW,), jnp.int32),
          pltpu.VMEM((HW,), jnp.int32),
          pltpu.VMEM((NBINS * LANES,), jnp.int32),
      ],
  )
  return f(sbits)


MASK_CBLK = 32


def _mask_body(x_ref, w_ref, o_ref):
  o_ref[0] = x_ref[0] * w_ref[...]


def _mask_mul(x, wmap_hw):
  return pl.pallas_call(
      _mask_body,
      grid=(N, C // MASK_CBLK),
      in_specs=[
          pl.BlockSpec((1, MASK_CBLK, H, W_DIM), lambda n, j: (n, j, 0, 0)),
          pl.BlockSpec((1, H, W_DIM), lambda n, j: (n, 0, 0)),
      ],
      out_specs=pl.BlockSpec((1, MASK_CBLK, H, W_DIM), lambda n, j: (n, j, 0, 0)),
      out_shape=jax.ShapeDtypeStruct((N, C, H, W_DIM), jnp.float32),
  )(x, wmap_hw)


def kernel(x, W1, b1, W2, b2):
  scores = _scores(x, W1, b1, W2, b2)
  sbits = lax.bitcast_convert_type(scores, jnp.int32).reshape(N * HW)
  tk_bits, w_bits = _topk(sbits)
  topk_vals = lax.bitcast_convert_type(tk_bits.reshape(N, K), jnp.float32)
  wmap_hw = lax.bitcast_convert_type(w_bits.reshape(N, H, W_DIM), jnp.float32)
  sparse = _mask_mul(x, wmap_hw)
  return sparse, topk_vals


# speedup vs baseline: 1.2161x; 1.1409x over previous
import functools
import math

import jax
import jax.numpy as jnp
from jax import lax
from jax.experimental import pallas as pl
from jax.experimental.pallas import tpu as pltpu
from jax.experimental.pallas import tpu_sc as plsc

N, C, H, W_DIM = 8, 256, 100, 252
HIDDEN = 256
HW = H * W_DIM
K = max(int(math.ceil(HW * 0.2)), 1)
LANES = 16
SEG = HW // LANES
NBINS = 256
SCORE_BLK = 2520
MASK_BLK = 3150


SCORE_HB = 8


def _score_body(x_ref, w1_ref, b1_ref, w2_ref, b2_ref, s_ref):
  w1 = w1_ref[...]
  w2 = w2_ref[...]
  for i in range(SCORE_HB):
    xs = x_ref[0, :, i, :]
    hdn = lax.dot_general(w1, xs, (((0,), (0,)), ((), ())),
                          preferred_element_type=jnp.float32)
    hdn = jnp.maximum(hdn + b1_ref[...], 0.0)
    z = lax.dot_general(w2, hdn, (((0,), (0,)), ((), ())),
                        preferred_element_type=jnp.float32)
    s_ref[0, pl.ds(i, 1), :] = jax.nn.sigmoid(z + b2_ref[...])


def _scores(x, W1, b1, W2, b2):
  return pl.pallas_call(
      _score_body,
      grid=(N, (H + SCORE_HB - 1) // SCORE_HB),
      in_specs=[
          pl.BlockSpec((1, C, SCORE_HB, W_DIM), lambda n, j: (n, 0, j, 0)),
          pl.BlockSpec((C, HIDDEN), lambda n, j: (0, 0)),
          pl.BlockSpec((HIDDEN, 1), lambda n, j: (0, 0)),
          pl.BlockSpec((HIDDEN, 1), lambda n, j: (0, 0)),
          pl.BlockSpec((1, 1), lambda n, j: (0, 0)),
      ],
      out_specs=pl.BlockSpec((1, SCORE_HB, W_DIM), lambda n, j: (n, j, 0)),
      out_shape=jax.ShapeDtypeStruct((N, H, W_DIM), jnp.float32),
  )(x, W1, b1.reshape(HIDDEN, 1), W2, b2.reshape(1, 1))


SC_CORES = 2
SC_SUBCORES = 16


def _sort_body(sbits_hbm, tk_hbm, wmap_hbm, ka, va, kb, vb, offs):
  wid = lax.axis_index("s") * SC_CORES + lax.axis_index("c")

  @pl.when(wid < N)
  def _():
    a = wid
    lane = lax.iota(jnp.int32, LANES)
    seg_base = lane * SEG
    ones16 = jnp.ones((LANES,), jnp.int32)
    zero16 = jnp.zeros((LANES,), jnp.int32)

    pltpu.sync_copy(sbits_hbm.at[pl.ds(a * HW, HW)], kb)

    def _init(t, _):
      sl = pl.ds(t * LANES, LANES)
      ka[sl] = ~kb[sl]
      va[sl] = t * LANES + lane
      return 0
    lax.fori_loop(0, HW // LANES, _init, 0)

    bufs = [(ka, va, kb, vb), (kb, vb, ka, va),
            (ka, va, kb, vb), (kb, vb, ka, va)]
    for p, (kin, vin, kout, vout) in enumerate(bufs):
      shift = 8 * p

      def _zero(i, _):
        offs[pl.ds(i * LANES, LANES)] = zero16
        return 0
      lax.fori_loop(0, NBINS, _zero, 0)

      def _hist(t, _, kin=kin, shift=shift):
        k16 = plsc.load_gather(kin, [seg_base + t])
        d = lax.shift_right_logical(k16, shift) & 0xFF
        plsc.addupdate_scatter(offs, [d * LANES + lane], ones16)
        return 0
      lax.fori_loop(0, SEG, _hist, 0)

      def _scan(d, run):
        sl = pl.ds(d * LANES, LANES)
        v = offs[sl]
        offs[sl] = run + plsc.cumsum(v) - v
        return run + jnp.sum(v)
      lax.fori_loop(0, NBINS, _scan, jnp.int32(0))

      def _place(t, _, kin=kin, vin=vin, kout=kout, vout=vout, shift=shift):
        idx = seg_base + t
        k16 = plsc.load_gather(kin, [idx])
        v16 = plsc.load_gather(vin, [idx])
        d = lax.shift_right_logical(k16, shift) & 0xFF
        addr = d * LANES + lane
        pos = plsc.load_gather(offs, [addr])
        plsc.store_scatter(kout, [pos], k16)
        plsc.store_scatter(vout, [pos], v16)
        plsc.addupdate_scatter(offs, [addr], ones16)
        return 0
      lax.fori_loop(0, SEG, _place, 0)

    def _vals(j, _):
      sl = pl.ds(j * LANES, LANES)
      kb[sl] = ~ka[sl]
      return 0
    lax.fori_loop(0, K // LANES, _vals, 0)
    pltpu.sync_copy(kb.at[pl.ds(0, K)], tk_hbm.at[pl.ds(a * K, K)])

    def _wzero(t, _):
      vb[pl.ds(t * LANES, LANES)] = zero16
      return 0
    lax.fori_loop(0, HW // LANES, _wzero, 0)

    def _wscat(j, _):
      sl = pl.ds(j * LANES, LANES)
      plsc.store_scatter(vb, [va[sl]], ~ka[sl])
      return 0
    lax.fori_loop(0, K // LANES, _wscat, 0)
    pltpu.sync_copy(vb, wmap_hbm.at[pl.ds(a * HW, HW)])


def _topk(sbits):
  mesh = plsc.VectorSubcoreMesh(core_axis_name="c", subcore_axis_name="s",
                                num_cores=SC_CORES, num_subcores=SC_SUBCORES)
  f = pl.kernel(
      _sort_body,
      out_type=(jax.ShapeDtypeStruct((N * K,), jnp.int32),
                jax.ShapeDtypeStruct((N * HW,), jnp.int32)),
      mesh=mesh,
      compiler_params=pltpu.CompilerParams(needs_layout_passes=False),
      scratch_types=[
          pltpu.VMEM((HW,), jnp.int32),
          pltpu.VMEM((HW,), jnp.int32),
          pltpu.VMEM((HW,), jnp.int32),
          pltpu.VMEM((HW,), jnp.int32),
          pltpu.VMEM((NBINS * LANES,), jnp.int32),
      ],
  )
  return f(sbits)


MASK_CBLK = 32


def _mask_body(x_ref, w_ref, o_ref):
  o_ref[0] = x_ref[0] * w_ref[...]


def _mask_mul(x, wmap_hw):
  return pl.pallas_call(
      _mask_body,
      grid=(N, C // MASK_CBLK),
      in_specs=[
          pl.BlockSpec((1, MASK_CBLK, H, W_DIM), lambda n, j: (n, j, 0, 0)),
          pl.BlockSpec((1, H, W_DIM), lambda n, j: (n, 0, 0)),
      ],
      out_specs=pl.BlockSpec((1, MASK_CBLK, H, W_DIM), lambda n, j: (n, j, 0, 0)),
      out_shape=jax.ShapeDtypeStruct((N, C, H, W_DIM), jnp.float32),
  )(x, wmap_hw)


def kernel(x, W1, b1, W2, b2):
  scores = _scores(x, W1, b1, W2, b2)
  sbits = lax.bitcast_convert_type(scores, jnp.int32).reshape(N * HW)
  tk_bits, w_bits = _topk(sbits)
  topk_vals = lax.bitcast_convert_type(tk_bits.reshape(N, K), jnp.float32)
  wmap_hw = lax.bitcast_convert_type(w_bits.reshape(N, H, W_DIM), jnp.float32)
  sparse = _mask_mul(x, wmap_hw)
  return sparse, topk_vals


# speedup vs baseline: 1.2620x; 1.0377x over previous
import functools
import math

import jax
import jax.numpy as jnp
from jax import lax
from jax.experimental import pallas as pl
from jax.experimental.pallas import tpu as pltpu
from jax.experimental.pallas import tpu_sc as plsc

N, C, H, W_DIM = 8, 256, 100, 252
HIDDEN = 256
HW = H * W_DIM
K = max(int(math.ceil(HW * 0.2)), 1)
LANES = 16
SEG = HW // LANES
NBINS = 1024
NPASS = 3
UNROLL = 5
SCORE_BLK = 2520
MASK_BLK = 3150


SCORE_HB = 8


def _score_body(x_ref, w1_ref, b1_ref, w2_ref, b2_ref, s_ref):
  w1 = w1_ref[...]
  w2 = w2_ref[...]
  for i in range(SCORE_HB):
    xs = x_ref[0, :, i, :]
    hdn = lax.dot_general(w1, xs, (((0,), (0,)), ((), ())),
                          preferred_element_type=jnp.float32)
    hdn = jnp.maximum(hdn + b1_ref[...], 0.0)
    z = lax.dot_general(w2, hdn, (((0,), (0,)), ((), ())),
                        preferred_element_type=jnp.float32)
    s_ref[0, pl.ds(i, 1), :] = jax.nn.sigmoid(z + b2_ref[...])


def _scores(x, W1, b1, W2, b2):
  return pl.pallas_call(
      _score_body,
      grid=(N, (H + SCORE_HB - 1) // SCORE_HB),
      in_specs=[
          pl.BlockSpec((1, C, SCORE_HB, W_DIM), lambda n, j: (n, 0, j, 0)),
          pl.BlockSpec((C, HIDDEN), lambda n, j: (0, 0)),
          pl.BlockSpec((HIDDEN, 1), lambda n, j: (0, 0)),
          pl.BlockSpec((HIDDEN, 1), lambda n, j: (0, 0)),
          pl.BlockSpec((1, 1), lambda n, j: (0, 0)),
      ],
      out_specs=pl.BlockSpec((1, SCORE_HB, W_DIM), lambda n, j: (n, j, 0)),
      out_shape=jax.ShapeDtypeStruct((N, H, W_DIM), jnp.float32),
  )(x, W1, b1.reshape(HIDDEN, 1), W2, b2.reshape(1, 1))


SC_CORES = 2
SC_SUBCORES = 16


def _sort_body(sbits_hbm, tk_hbm, wmap_hbm, ka, va, kb, vb, offs):
  wid = lax.axis_index("s") * SC_CORES + lax.axis_index("c")

  @pl.when(wid < N)
  def _():
    a = wid
    lane = lax.iota(jnp.int32, LANES)
    seg_base = lane * SEG
    ones16 = jnp.ones((LANES,), jnp.int32)
    zero16 = jnp.zeros((LANES,), jnp.int32)

    pltpu.sync_copy(sbits_hbm.at[pl.ds(a * HW, HW)], kb)

    def _init(t, _):
      for j in range(UNROLL):
        sl = pl.ds((t * UNROLL + j) * LANES, LANES)
        ka[sl] = ~kb[sl]
        va[sl] = (t * UNROLL + j) * LANES + lane
      return 0
    lax.fori_loop(0, HW // LANES // UNROLL, _init, 0)

    bufs = [(ka, va, kb, vb), (kb, vb, ka, va), (ka, va, kb, vb)]
    for p, (kin, vin, kout, vout) in enumerate(bufs):
      shift = 10 * p

      def _zero(i, _):
        for j in range(4):
          offs[pl.ds((i * 4 + j) * LANES, LANES)] = zero16
        return 0
      lax.fori_loop(0, NBINS // 4, _zero, 0)

      def _hist(t, _, kin=kin, shift=shift):
        for j in range(UNROLL):
          k16 = plsc.load_gather(kin, [seg_base + (t * UNROLL + j)])
          d = lax.shift_right_logical(k16, shift) & 0x3FF
          plsc.addupdate_scatter(offs, [d * LANES + lane], ones16)
        return 0
      lax.fori_loop(0, SEG // UNROLL, _hist, 0)

      def _scan(i, run):
        for j in range(4):
          sl = pl.ds((i * 4 + j) * LANES, LANES)
          v = offs[sl]
          offs[sl] = run + plsc.cumsum(v) - v
          run = run + jnp.sum(v)
        return run
      lax.fori_loop(0, NBINS // 4, _scan, jnp.int32(0))

      def _place(t, _, kin=kin, vin=vin, kout=kout, vout=vout, shift=shift):
        for j in range(UNROLL):
          idx = seg_base + (t * UNROLL + j)
          k16 = plsc.load_gather(kin, [idx])
          v16 = plsc.load_gather(vin, [idx])
          d = lax.shift_right_logical(k16, shift) & 0x3FF
          addr = d * LANES + lane
          pos = plsc.load_gather(offs, [addr])
          plsc.store_scatter(kout, [pos], k16)
          plsc.store_scatter(vout, [pos], v16)
          plsc.addupdate_scatter(offs, [addr], ones16)
        return 0
      lax.fori_loop(0, SEG // UNROLL, _place, 0)

    def _vals(j, _):
      for i in range(UNROLL):
        sl = pl.ds((j * UNROLL + i) * LANES, LANES)
        ka[sl] = ~kb[sl]
      return 0
    lax.fori_loop(0, K // LANES // UNROLL, _vals, 0)
    pltpu.sync_copy(ka.at[pl.ds(0, K)], tk_hbm.at[pl.ds(a * K, K)])

    def _wzero(t, _):
      for j in range(UNROLL):
        va[pl.ds((t * UNROLL + j) * LANES, LANES)] = zero16
      return 0
    lax.fori_loop(0, HW // LANES // UNROLL, _wzero, 0)

    def _wscat(j, _):
      for i in range(UNROLL):
        sl = pl.ds((j * UNROLL + i) * LANES, LANES)
        plsc.store_scatter(va, [vb[sl]], ~kb[sl])
      return 0
    lax.fori_loop(0, K // LANES // UNROLL, _wscat, 0)
    pltpu.sync_copy(va, wmap_hbm.at[pl.ds(a * HW, HW)])


def _topk(sbits):
  mesh = plsc.VectorSubcoreMesh(core_axis_name="c", subcore_axis_name="s",
                                num_cores=SC_CORES, num_subcores=SC_SUBCORES)
  f = pl.kernel(
      _sort_body,
      out_type=(jax.ShapeDtypeStruct((N * K,), jnp.int32),
                jax.ShapeDtypeStruct((N * HW,), jnp.int32)),
      mesh=mesh,
      compiler_params=pltpu.CompilerParams(needs_layout_passes=False),
      scratch_types=[
          pltpu.VMEM((HW,), jnp.int32),
          pltpu.VMEM((HW,), jnp.int32),
          pltpu.VMEM((HW,), jnp.int32),
          pltpu.VMEM((HW,), jnp.int32),
          pltpu.VMEM((NBINS * LANES,), jnp.int32),
      ],
  )
  return f(sbits)


MASK_CBLK = 32


def _mask_body(x_ref, w_ref, o_ref):
  o_ref[0] = x_ref[0] * w_ref[...]


def _mask_mul(x, wmap_hw):
  return pl.pallas_call(
      _mask_body,
      grid=(N, C // MASK_CBLK),
      in_specs=[
          pl.BlockSpec((1, MASK_CBLK, H, W_DIM), lambda n, j: (n, j, 0, 0)),
          pl.BlockSpec((1, H, W_DIM), lambda n, j: (n, 0, 0)),
      ],
      out_specs=pl.BlockSpec((1, MASK_CBLK, H, W_DIM), lambda n, j: (n, j, 0, 0)),
      out_shape=jax.ShapeDtypeStruct((N, C, H, W_DIM), jnp.float32),
  )(x, wmap_hw)


def kernel(x, W1, b1, W2, b2):
  scores = _scores(x, W1, b1, W2, b2)
  sbits = lax.bitcast_convert_type(scores, jnp.int32).reshape(N * HW)
  tk_bits, w_bits = _topk(sbits)
  topk_vals = lax.bitcast_convert_type(tk_bits.reshape(N, K), jnp.float32)
  wmap_hw = lax.bitcast_convert_type(w_bits.reshape(N, H, W_DIM), jnp.float32)
  sparse = _mask_mul(x, wmap_hw)
  return sparse, topk_vals
